# baseline (speedup 1.0000x reference)
import jax
import jax.numpy as jnp
from jax import lax
from jax.experimental import pallas as pl
from jax.experimental.pallas import tpu as pltpu
from jax.experimental.pallas import tpu_sc as plsc

N_NODE = 10000
N_EDGE = 320000
D = 128

_SC_INFO = plsc.get_sparse_core_info()
NUM_CORES = _SC_INFO.num_cores
NUM_SUBCORES = _SC_INFO.num_subcores

EDGES_PER_TILE = N_EDGE // NUM_SUBCORES
CHUNK = 40
CHUNKS_PER_SUPER = 50
EDGES_PER_SUPER = CHUNK * CHUNKS_PER_SUPER
N_SUPER = EDGES_PER_TILE // EDGES_PER_SUPER

ROWS_PER_WB = 624
WB_TAIL = N_NODE - NUM_SUBCORES * ROWS_PER_WB

DEG_CHUNK = 2000


def _sc_body(x_user, x_item, ef0, ef1, zeros_hbm,
             agg_item, deg_item, agg_user, deg_user,
             acc_sh, deg_sh, sidx0, sidx1, didx0, didx1, rows, ones_v, zdeg,
             semg0, semg1, sems0, sems1, semi):
  core = lax.axis_index("c")
  sid = lax.axis_index("s")
  semg = (semg0, semg1)
  sems = (sems0, sems1)
  sidxs = (sidx0, sidx1)
  didxs = (didx0, didx1)

  def fire_prefetch(ef_hbm, s, bank):
    base = sid * EDGES_PER_TILE + s * EDGES_PER_SUPER
    pltpu.async_copy(ef_hbm.at[pl.ds(base, EDGES_PER_SUPER)],
                     sidxs[bank], semi)
    pltpu.async_copy(ef_hbm.at[pl.ds(N_EDGE + base, EDGES_PER_SUPER)],
                     didxs[bank], semi)

  def wait_prefetch(ef_hbm, bank):
    pltpu.make_async_copy(ef_hbm.at[pl.ds(0, EDGES_PER_SUPER)],
                          sidxs[bank], semi).wait()
    pltpu.make_async_copy(ef_hbm.at[pl.ds(0, EDGES_PER_SUPER)],
                          didxs[bank], semi).wait()

  @pl.when(core == 0)
  def _():
    fire_prefetch(ef0, 0, 0)

  @pl.when(core == 1)
  def _():
    fire_prefetch(ef1, 0, 0)

  zvec = jnp.zeros((16,), jnp.float32)

  @pl.loop(0, DEG_CHUNK // 16)
  def _(i):
    zdeg[pl.ds(i * 16, 16)] = zvec

  for j in range(8):
    ones_v[pl.ds(j * 16, 16)] = jnp.ones((16,), jnp.float32)

  pltpu.sync_copy(zeros_hbm.at[pl.ds(0, ROWS_PER_WB)],
                  acc_sh.at[pl.ds(sid * ROWS_PER_WB, ROWS_PER_WB)])

  @pl.when(sid == 0)
  def _():
    pltpu.sync_copy(zeros_hbm.at[pl.ds(0, WB_TAIL)],
                    acc_sh.at[pl.ds(NUM_SUBCORES * ROWS_PER_WB, WB_TAIL)])

  @pl.when(sid < N_NODE // DEG_CHUNK)
  def _():
    pltpu.sync_copy(zdeg, deg_sh.at[pl.ds(sid * DEG_CHUNK, DEG_CHUNK)])

  plsc.subcore_barrier()

  def run_relation(x_hbm, ef_hbm):
    def idx_slice(buf, j):
      off = pl.multiple_of(j * CHUNK, 8)
      return buf.at[pl.ds(off, CHUNK)]

    def fire_gather(bank, sidx_b, j):
      return pltpu.async_copy(x_hbm.at[idx_slice(sidx_b, j)],
                              rows.at[bank], semg[bank])

    def fire_scatter(bank, didx_b, j):
      pltpu.async_copy(rows.at[bank], acc_sh.at[idx_slice(didx_b, j)],
                       sems[bank], add=True)
      pltpu.async_copy(ones_v.at[pl.ds(0, CHUNK)],
                       deg_sh.at[idx_slice(didx_b, j)],
                       sems[bank], add=True)

    def drain_scatter(bank):
      pltpu.make_async_copy(rows.at[bank], acc_sh.at[idx_slice(didx0, 0)],
                            sems[bank]).wait()
      pltpu.make_async_copy(ones_v.at[pl.ds(0, CHUNK)],
                            deg_sh.at[idx_slice(didx0, 0)],
                            sems[bank]).wait()

    @pl.loop(0, N_SUPER // 2)
    def _(s2):
      for half in (0, 1):
        sup = 2 * s2 + half

        @pl.when(sup > 0)
        def _():
          drain_scatter(0)
          drain_scatter(1)

        wait_prefetch(ef_hbm, half)

        @pl.when(sup < N_SUPER - 1)
        def _():
          fire_prefetch(ef_hbm, sup + 1, 1 - half)

        sidx_b, didx_b = sidxs[half], didxs[half]

        @pl.loop(0, CHUNKS_PER_SUPER // 2)
        def _(g):
          @pl.when(g > 0)
          def _():
            drain_scatter(0)
            drain_scatter(1)

          ga = fire_gather(0, sidx_b, 2 * g)
          gb = fire_gather(1, sidx_b, 2 * g + 1)
          ga.wait()
          fire_scatter(0, didx_b, 2 * g)
          gb.wait()
          fire_scatter(1, didx_b, 2 * g + 1)

    drain_scatter(0)
    drain_scatter(1)

  @pl.when(core == 0)
  def _():
    run_relation(x_user, ef0)

  @pl.when(core == 1)
  def _():
    run_relation(x_item, ef1)

  plsc.subcore_barrier()

  def writeback(agg_out, deg_out):
    pltpu.sync_copy(acc_sh.at[pl.ds(sid * ROWS_PER_WB, ROWS_PER_WB)],
                    agg_out.at[pl.ds(sid * ROWS_PER_WB, ROWS_PER_WB)])

    @pl.when(sid == 0)
    def _():
      pltpu.sync_copy(acc_sh.at[pl.ds(NUM_SUBCORES * ROWS_PER_WB, WB_TAIL)],
                      agg_out.at[pl.ds(NUM_SUBCORES * ROWS_PER_WB, WB_TAIL)])

    @pl.when(sid < N_NODE // DEG_CHUNK)
    def _():
      pltpu.sync_copy(deg_sh.at[pl.ds(sid * DEG_CHUNK, DEG_CHUNK)], zdeg)
      pltpu.sync_copy(zdeg, deg_out.at[pl.ds(sid * DEG_CHUNK, DEG_CHUNK)])

  @pl.when(core == 0)
  def _():
    writeback(agg_item, deg_item)

  @pl.when(core == 1)
  def _():
    writeback(agg_user, deg_user)


_sc_aggregate = pl.kernel(
    _sc_body,
    out_type=(
        jax.ShapeDtypeStruct((N_NODE, D), jnp.float32),
        jax.ShapeDtypeStruct((N_NODE,), jnp.float32),
        jax.ShapeDtypeStruct((N_NODE, D), jnp.float32),
        jax.ShapeDtypeStruct((N_NODE,), jnp.float32),
    ),
    mesh=plsc.VectorSubcoreMesh(core_axis_name="c", subcore_axis_name="s"),
    scratch_types=[
        pltpu.VMEM_SHARED((N_NODE, D), jnp.float32),
        pltpu.VMEM_SHARED((N_NODE,), jnp.float32),
        pltpu.VMEM((EDGES_PER_SUPER,), jnp.int32),
        pltpu.VMEM((EDGES_PER_SUPER,), jnp.int32),
        pltpu.VMEM((EDGES_PER_SUPER,), jnp.int32),
        pltpu.VMEM((EDGES_PER_SUPER,), jnp.int32),
        pltpu.VMEM((2, CHUNK, D), jnp.float32),
        pltpu.VMEM((128,), jnp.float32),
        pltpu.VMEM((DEG_CHUNK,), jnp.float32),
        pltpu.SemaphoreType.DMA,
        pltpu.SemaphoreType.DMA,
        pltpu.SemaphoreType.DMA,
        pltpu.SemaphoreType.DMA,
        pltpu.SemaphoreType.DMA,
    ],
)


ROW_BLK = 1000


def _tc_body(agg_i, deg_i, W0, b0, agg_u, deg_u, W1, b1, out_i, out_u):
  hi = jnp.dot(agg_i[...], W0[...], preferred_element_type=jnp.float32,
               precision=lax.Precision.HIGHEST)
  out_i[...] = jnp.maximum(hi + deg_i[...] * b0[...], 0.0)
  hu = jnp.dot(agg_u[...], W1[...], preferred_element_type=jnp.float32,
               precision=lax.Precision.HIGHEST)
  out_u[...] = jnp.maximum(hu + deg_u[...] * b1[...], 0.0)


_tc_epilogue = pl.pallas_call(
    _tc_body,
    grid=(N_NODE // ROW_BLK,),
    in_specs=[
        pl.BlockSpec((ROW_BLK, D), lambda i: (i, 0)),
        pl.BlockSpec((ROW_BLK, 1), lambda i: (i, 0)),
        pl.BlockSpec((D, D), lambda i: (0, 0)),
        pl.BlockSpec((1, D), lambda i: (0, 0)),
        pl.BlockSpec((ROW_BLK, D), lambda i: (i, 0)),
        pl.BlockSpec((ROW_BLK, 1), lambda i: (i, 0)),
        pl.BlockSpec((D, D), lambda i: (0, 0)),
        pl.BlockSpec((1, D), lambda i: (0, 0)),
    ],
    out_specs=[
        pl.BlockSpec((ROW_BLK, D), lambda i: (i, 0)),
        pl.BlockSpec((ROW_BLK, D), lambda i: (i, 0)),
    ],
    out_shape=[
        jax.ShapeDtypeStruct((N_NODE, D), jnp.float32),
        jax.ShapeDtypeStruct((N_NODE, D), jnp.float32),
    ],
)


def kernel(x_user, x_item, W_rel0, b_rel0, W_rel1, b_rel1,
           edge_index_rel0, edge_index_rel1):
  ef0 = edge_index_rel0.astype(jnp.int32).reshape(2 * N_EDGE)
  ef1 = edge_index_rel1.astype(jnp.int32).reshape(2 * N_EDGE)

  zeros_hbm = jnp.zeros((ROWS_PER_WB, D), jnp.float32)
  agg_item, deg_item, agg_user, deg_user = _sc_aggregate(
      x_user, x_item, ef0, ef1, zeros_hbm)

  h_item, h_user = _tc_epilogue(
      agg_item, deg_item.reshape(N_NODE, 1), W_rel0, b_rel0.reshape(1, D),
      agg_user, deg_user.reshape(N_NODE, 1), W_rel1, b_rel1.reshape(1, D))
  return (h_user, h_item)


# speedup vs baseline: 1.2181x; 1.2181x over previous
import jax
import jax.numpy as jnp
from jax import lax
from jax.experimental import pallas as pl
from jax.experimental.pallas import tpu as pltpu
from jax.experimental.pallas import tpu_sc as plsc

N_NODE = 10000
N_EDGE = 320000
D = 128

_SC_INFO = plsc.get_sparse_core_info()
NUM_CORES = _SC_INFO.num_cores
NUM_SUBCORES = _SC_INFO.num_subcores

EDGES_PER_TILE = N_EDGE // NUM_SUBCORES
CHUNK = 125
CHUNKS_PER_SUPER = 16
EDGES_PER_SUPER = CHUNK * CHUNKS_PER_SUPER
N_SUPER = EDGES_PER_TILE // EDGES_PER_SUPER

ROWS_PER_WB = 624
WB_TAIL = N_NODE - NUM_SUBCORES * ROWS_PER_WB

DEG_CHUNK = 2000


def _sc_body(x_user, x_item, er0, er1, zeros_hbm,
             agg_item, deg_item, agg_user, deg_user,
             acc_sh, deg_sh, sidx, didx, rows, ones_v, zdeg,
             semg0, semg1, sems0, sems1, semi):
  core = lax.axis_index("c")
  sid = lax.axis_index("s")
  semg = (semg0, semg1)
  sems = (sems0, sems1)

  def fire_prefetch(er_hbm, s, bank):
    sup = sid * N_SUPER + s
    pltpu.async_copy(er_hbm.at[0, sup], sidx.at[bank], semi)
    pltpu.async_copy(er_hbm.at[1, sup], didx.at[bank], semi)

  def wait_prefetch(er_hbm, bank):
    pltpu.make_async_copy(er_hbm.at[0, 0], sidx.at[bank], semi).wait()
    pltpu.make_async_copy(er_hbm.at[1, 0], didx.at[bank], semi).wait()

  @pl.when(core == 0)
  def _():
    fire_prefetch(er0, 0, 0)

  @pl.when(core == 1)
  def _():
    fire_prefetch(er1, 0, 0)

  zvec = jnp.zeros((16,), jnp.float32)

  @pl.loop(0, DEG_CHUNK // 16)
  def _(i):
    zdeg[pl.ds(i * 16, 16)] = zvec

  for j in range(8):
    ones_v[pl.ds(j * 16, 16)] = jnp.ones((16,), jnp.float32)

  pltpu.sync_copy(zeros_hbm.at[pl.ds(0, ROWS_PER_WB)],
                  acc_sh.at[pl.ds(sid * ROWS_PER_WB, ROWS_PER_WB)])

  @pl.when(sid == 0)
  def _():
    pltpu.sync_copy(zeros_hbm.at[pl.ds(0, WB_TAIL)],
                    acc_sh.at[pl.ds(NUM_SUBCORES * ROWS_PER_WB, WB_TAIL)])

  @pl.when(sid < N_NODE // DEG_CHUNK)
  def _():
    pltpu.sync_copy(zdeg, deg_sh.at[pl.ds(sid * DEG_CHUNK, DEG_CHUNK)])

  plsc.subcore_barrier()

  def run_relation(x_hbm, er_hbm):
    def fire_gather(bank, ib, j):
      return pltpu.async_copy(x_hbm.at[sidx.at[ib, j]], rows.at[bank],
                              semg[bank])

    def fire_scatter(bank, ib, j):
      pltpu.async_copy(rows.at[bank], acc_sh.at[didx.at[ib, j]], sems[bank],
                       add=True)
      pltpu.async_copy(ones_v.at[pl.ds(0, CHUNK)], deg_sh.at[didx.at[ib, j]],
                       sems[bank], add=True)

    def drain_scatter(bank):
      pltpu.make_async_copy(rows.at[bank], acc_sh.at[didx.at[0, 0]],
                            sems[bank]).wait()
      pltpu.make_async_copy(ones_v.at[pl.ds(0, CHUNK)],
                            deg_sh.at[didx.at[0, 0]], sems[bank]).wait()

    @pl.loop(0, N_SUPER)
    def _(s):
      ib = lax.rem(s, 2)

      @pl.when(s > 0)
      def _():
        drain_scatter(0)
        drain_scatter(1)

      wait_prefetch(er_hbm, ib)

      @pl.when(s < N_SUPER - 1)
      def _():
        fire_prefetch(er_hbm, s + 1, 1 - ib)

      @pl.loop(0, CHUNKS_PER_SUPER // 2)
      def _(g):
        @pl.when(g > 0)
        def _():
          drain_scatter(0)
          drain_scatter(1)

        ga = fire_gather(0, ib, 2 * g)
        gb = fire_gather(1, ib, 2 * g + 1)
        ga.wait()
        fire_scatter(0, ib, 2 * g)
        gb.wait()
        fire_scatter(1, ib, 2 * g + 1)

    drain_scatter(0)
    drain_scatter(1)

  @pl.when(core == 0)
  def _():
    run_relation(x_user, er0)

  @pl.when(core == 1)
  def _():
    run_relation(x_item, er1)

  plsc.subcore_barrier()

  def writeback(agg_out, deg_out):
    pltpu.sync_copy(acc_sh.at[pl.ds(sid * ROWS_PER_WB, ROWS_PER_WB)],
                    agg_out.at[pl.ds(sid * ROWS_PER_WB, ROWS_PER_WB)])

    @pl.when(sid == 0)
    def _():
      pltpu.sync_copy(acc_sh.at[pl.ds(NUM_SUBCORES * ROWS_PER_WB, WB_TAIL)],
                      agg_out.at[pl.ds(NUM_SUBCORES * ROWS_PER_WB, WB_TAIL)])

    @pl.when(sid < N_NODE // DEG_CHUNK)
    def _():
      pltpu.sync_copy(deg_sh.at[pl.ds(sid * DEG_CHUNK, DEG_CHUNK)], zdeg)
      pltpu.sync_copy(zdeg, deg_out.at[pl.ds(sid * DEG_CHUNK, DEG_CHUNK)])

  @pl.when(core == 0)
  def _():
    writeback(agg_item, deg_item)

  @pl.when(core == 1)
  def _():
    writeback(agg_user, deg_user)


_sc_aggregate = pl.kernel(
    _sc_body,
    out_type=(
        jax.ShapeDtypeStruct((N_NODE, D), jnp.float32),
        jax.ShapeDtypeStruct((N_NODE,), jnp.float32),
        jax.ShapeDtypeStruct((N_NODE, D), jnp.float32),
        jax.ShapeDtypeStruct((N_NODE,), jnp.float32),
    ),
    mesh=plsc.VectorSubcoreMesh(core_axis_name="c", subcore_axis_name="s"),
    scratch_types=[
        pltpu.VMEM_SHARED((N_NODE, D), jnp.float32),
        pltpu.VMEM_SHARED((N_NODE,), jnp.float32),
        pltpu.VMEM((2, CHUNKS_PER_SUPER, CHUNK), jnp.int32),
        pltpu.VMEM((2, CHUNKS_PER_SUPER, CHUNK), jnp.int32),
        pltpu.VMEM((2, CHUNK, D), jnp.float32),
        pltpu.VMEM((128,), jnp.float32),
        pltpu.VMEM((DEG_CHUNK,), jnp.float32),
        pltpu.SemaphoreType.DMA,
        pltpu.SemaphoreType.DMA,
        pltpu.SemaphoreType.DMA,
        pltpu.SemaphoreType.DMA,
        pltpu.SemaphoreType.DMA,
    ],
)


ROW_BLK = 1000


def _tc_body(agg_i, deg_i, W0, b0, agg_u, deg_u, W1, b1, out_i, out_u):
  hi = jnp.dot(agg_i[...], W0[...], preferred_element_type=jnp.float32,
               precision=lax.Precision.HIGHEST)
  out_i[...] = jnp.maximum(hi + deg_i[...] * b0[...], 0.0)
  hu = jnp.dot(agg_u[...], W1[...], preferred_element_type=jnp.float32,
               precision=lax.Precision.HIGHEST)
  out_u[...] = jnp.maximum(hu + deg_u[...] * b1[...], 0.0)


_tc_epilogue = pl.pallas_call(
    _tc_body,
    grid=(N_NODE // ROW_BLK,),
    in_specs=[
        pl.BlockSpec((ROW_BLK, D), lambda i: (i, 0)),
        pl.BlockSpec((ROW_BLK, 1), lambda i: (i, 0)),
        pl.BlockSpec((D, D), lambda i: (0, 0)),
        pl.BlockSpec((1, D), lambda i: (0, 0)),
        pl.BlockSpec((ROW_BLK, D), lambda i: (i, 0)),
        pl.BlockSpec((ROW_BLK, 1), lambda i: (i, 0)),
        pl.BlockSpec((D, D), lambda i: (0, 0)),
        pl.BlockSpec((1, D), lambda i: (0, 0)),
    ],
    out_specs=[
        pl.BlockSpec((ROW_BLK, D), lambda i: (i, 0)),
        pl.BlockSpec((ROW_BLK, D), lambda i: (i, 0)),
    ],
    out_shape=[
        jax.ShapeDtypeStruct((N_NODE, D), jnp.float32),
        jax.ShapeDtypeStruct((N_NODE, D), jnp.float32),
    ],
)


def kernel(x_user, x_item, W_rel0, b_rel0, W_rel1, b_rel1,
           edge_index_rel0, edge_index_rel1):
  er_shape = (2, N_EDGE // EDGES_PER_SUPER, CHUNKS_PER_SUPER, CHUNK)
  er0 = edge_index_rel0.astype(jnp.int32).reshape(er_shape)
  er1 = edge_index_rel1.astype(jnp.int32).reshape(er_shape)

  zeros_hbm = jnp.zeros((ROWS_PER_WB, D), jnp.float32)
  agg_item, deg_item, agg_user, deg_user = _sc_aggregate(
      x_user, x_item, er0, er1, zeros_hbm)

  h_item, h_user = _tc_epilogue(
      agg_item, deg_item.reshape(N_NODE, 1), W_rel0, b_rel0.reshape(1, D),
      agg_user, deg_user.reshape(N_NODE, 1), W_rel1, b_rel1.reshape(1, D))
  return (h_user, h_item)
